# baseline (speedup 1.0000x reference)
import jax
import jax.numpy as jnp
from jax.experimental import pallas as pl

N_NODES = 100000
HIDDEN = 64
NUM_GRAPHS = 128
BLK = 1000


def _lin_kernel(x_ref, w_ref, o_ref):
    o_ref[...] = jnp.dot(x_ref[...], w_ref[...],
                         preferred_element_type=jnp.float32)


def _fused_kernel(a_ref, b_ref, w_ref, o_ref):
    h = jnp.maximum(a_ref[...] + b_ref[...], 0.0)
    o_ref[...] = jnp.dot(h, w_ref[...], preferred_element_type=jnp.float32)


def _pool_kernel(a_ref, b_ref, seg_ref, sums_ref, cnt_ref):
    i = pl.program_id(0)

    @pl.when(i == 0)
    def _():
        sums_ref[...] = jnp.zeros_like(sums_ref)
        cnt_ref[...] = jnp.zeros_like(cnt_ref)

    h = jnp.maximum(a_ref[...] + b_ref[...], 0.0)
    seg = seg_ref[...]
    gids = jax.lax.broadcasted_iota(jnp.int32, (BLK, NUM_GRAPHS), 1)
    onehot = (seg == gids).astype(jnp.float32)
    sums_ref[...] += jax.lax.dot_general(
        onehot, h, (((0,), (0,)), ((), ())),
        preferred_element_type=jnp.float32)
    cnt_ref[...] += jnp.sum(onehot, axis=0, keepdims=True)


def _linear(x, W):
    n, k = x.shape
    return pl.pallas_call(
        _lin_kernel,
        grid=(n // BLK,),
        in_specs=[
            pl.BlockSpec((BLK, k), lambda i: (i, 0)),
            pl.BlockSpec((k, HIDDEN), lambda i: (0, 0)),
        ],
        out_specs=pl.BlockSpec((BLK, HIDDEN), lambda i: (i, 0)),
        out_shape=jax.ShapeDtypeStruct((n, HIDDEN), jnp.float32),
    )(x, W)


def _fused(a, b, W):
    n = a.shape[0]
    return pl.pallas_call(
        _fused_kernel,
        grid=(n // BLK,),
        in_specs=[
            pl.BlockSpec((BLK, HIDDEN), lambda i: (i, 0)),
            pl.BlockSpec((1, HIDDEN), lambda i: (0, 0)),
            pl.BlockSpec((HIDDEN, HIDDEN), lambda i: (0, 0)),
        ],
        out_specs=pl.BlockSpec((BLK, HIDDEN), lambda i: (i, 0)),
        out_shape=jax.ShapeDtypeStruct((n, HIDDEN), jnp.float32),
    )(a, b.reshape(1, HIDDEN), W)


def _pool(a, b, seg):
    n = a.shape[0]
    sums, cnt = pl.pallas_call(
        _pool_kernel,
        grid=(n // BLK,),
        in_specs=[
            pl.BlockSpec((BLK, HIDDEN), lambda i: (i, 0)),
            pl.BlockSpec((1, HIDDEN), lambda i: (0, 0)),
            pl.BlockSpec((BLK, 1), lambda i: (i, 0)),
        ],
        out_specs=[
            pl.BlockSpec((NUM_GRAPHS, HIDDEN), lambda i: (0, 0)),
            pl.BlockSpec((1, NUM_GRAPHS), lambda i: (0, 0)),
        ],
        out_shape=[
            jax.ShapeDtypeStruct((NUM_GRAPHS, HIDDEN), jnp.float32),
            jax.ShapeDtypeStruct((1, NUM_GRAPHS), jnp.float32),
        ],
    )(a, b.reshape(1, HIDDEN), seg)
    return sums, cnt[0]


def kernel(x, edge_index, batch, W1, b1, W2, b2, W3, b3, W4, b4):
    N = x.shape[0]
    ei = edge_index.astype(jnp.int32)
    loop = jnp.arange(N, dtype=jnp.int32)
    src = jnp.concatenate([ei[0], loop])
    dst = jnp.concatenate([ei[1], loop])
    deg = jnp.zeros((N,), jnp.float32).at[dst].add(1.0)
    dinv = jax.lax.rsqrt(jnp.maximum(deg, 1.0))
    coef = (dinv[src] * dinv[dst])[:, None]

    def agg(h):
        msg = h[src] * coef
        return jnp.zeros((N, HIDDEN), jnp.float32).at[dst].add(msg)

    a = agg(_linear(x, W1))
    a = agg(_fused(a, b1, W2))
    a = agg(_fused(a, b2, W3))
    a = agg(_fused(a, b3, W4))

    seg = batch.astype(jnp.int32).reshape(N, 1)
    sums, cnt = _pool(a, b4, seg)
    return sums / jnp.maximum(cnt, 1.0)[:, None]


# speedup vs baseline: 1.1050x; 1.1050x over previous
import jax
import jax.numpy as jnp
from jax.experimental import pallas as pl

N_NODES = 100000
HIDDEN = 64
NUM_GRAPHS = 128
BLK = 1000


def _lin_kernel(x_ref, w_ref, o_ref):
    o_ref[...] = jnp.dot(x_ref[...], w_ref[...],
                         preferred_element_type=jnp.float32)


def _fused_kernel(a_ref, b_ref, w_ref, o_ref):
    h = jnp.maximum(a_ref[...] + b_ref[...], 0.0)
    o_ref[...] = jnp.dot(h, w_ref[...], preferred_element_type=jnp.float32)


def _pool_kernel(a_ref, b_ref, seg_ref, sums_ref, cnt_ref):
    i = pl.program_id(0)

    @pl.when(i == 0)
    def _():
        sums_ref[...] = jnp.zeros_like(sums_ref)
        cnt_ref[...] = jnp.zeros_like(cnt_ref)

    h = jnp.maximum(a_ref[...] + b_ref[...], 0.0)
    seg = seg_ref[...]
    gids = jax.lax.broadcasted_iota(jnp.int32, (BLK, NUM_GRAPHS), 1)
    onehot = (seg == gids).astype(jnp.float32)
    sums_ref[...] += jax.lax.dot_general(
        onehot, h, (((0,), (0,)), ((), ())),
        preferred_element_type=jnp.float32)
    cnt_ref[...] += jnp.sum(onehot, axis=0, keepdims=True)


def _linear(x, W):
    n, k = x.shape
    return pl.pallas_call(
        _lin_kernel,
        grid=(n // BLK,),
        in_specs=[
            pl.BlockSpec((BLK, k), lambda i: (i, 0)),
            pl.BlockSpec((k, HIDDEN), lambda i: (0, 0)),
        ],
        out_specs=pl.BlockSpec((BLK, HIDDEN), lambda i: (i, 0)),
        out_shape=jax.ShapeDtypeStruct((n, HIDDEN), jnp.float32),
    )(x, W)


def _fused(a, b, W):
    n = a.shape[0]
    return pl.pallas_call(
        _fused_kernel,
        grid=(n // BLK,),
        in_specs=[
            pl.BlockSpec((BLK, HIDDEN), lambda i: (i, 0)),
            pl.BlockSpec((1, HIDDEN), lambda i: (0, 0)),
            pl.BlockSpec((HIDDEN, HIDDEN), lambda i: (0, 0)),
        ],
        out_specs=pl.BlockSpec((BLK, HIDDEN), lambda i: (i, 0)),
        out_shape=jax.ShapeDtypeStruct((n, HIDDEN), jnp.float32),
    )(a, b.reshape(1, HIDDEN), W)


def _pool(a, b, seg):
    n = a.shape[0]
    sums, cnt = pl.pallas_call(
        _pool_kernel,
        grid=(n // BLK,),
        in_specs=[
            pl.BlockSpec((BLK, HIDDEN), lambda i: (i, 0)),
            pl.BlockSpec((1, HIDDEN), lambda i: (0, 0)),
            pl.BlockSpec((BLK, 1), lambda i: (i, 0)),
        ],
        out_specs=[
            pl.BlockSpec((NUM_GRAPHS, HIDDEN), lambda i: (0, 0)),
            pl.BlockSpec((1, NUM_GRAPHS), lambda i: (0, 0)),
        ],
        out_shape=[
            jax.ShapeDtypeStruct((NUM_GRAPHS, HIDDEN), jnp.float32),
            jax.ShapeDtypeStruct((1, NUM_GRAPHS), jnp.float32),
        ],
    )(a, b.reshape(1, HIDDEN), seg)
    return sums, cnt[0]


def kernel(x, edge_index, batch, W1, b1, W2, b2, W3, b3, W4, b4):
    N = x.shape[0]
    ei = edge_index.astype(jnp.int32)
    src = ei[0]
    dst = ei[1]
    deg = jnp.zeros((N,), jnp.float32).at[dst].add(1.0) + 1.0
    dinv = jax.lax.rsqrt(deg)
    coef = (dinv[src] * dinv[dst])[:, None]
    selfc = (dinv * dinv)[:, None]

    def agg(h):
        msg = h[src] * coef
        return (h * selfc).at[dst].add(msg)

    a = agg(_linear(x, W1))
    a = agg(_fused(a, b1, W2))
    a = agg(_fused(a, b2, W3))
    a = agg(_fused(a, b3, W4))

    seg = batch.astype(jnp.int32).reshape(N, 1)
    sums, cnt = _pool(a, b4, seg)
    return sums / jnp.maximum(cnt, 1.0)[:, None]
